# baseline (speedup 1.0000x reference)
import functools

import jax
import jax.numpy as jnp
from jax.experimental import pallas as pl
from jax.experimental.pallas import tpu as pltpu

_THRESHOLD = 1.5


def _nn_body(q_ref, k_ref, idx_ref, min_ref, sval, sbase, sqsq, *, bk, nb):
    i = pl.program_id(0)
    q = q_ref[...]
    k = k_ref[...]
    nq = q.shape[0]

    @pl.when(i == 0)
    def _qsq():
        sqsq[...] = jnp.sum(q * q, axis=1, keepdims=True)

    q_sq = sqsq[...]

    val = jnp.where(i == 0, jnp.inf, sval[...])
    base = jnp.where(i == 0, jnp.float32(0), sbase[...])
    off = (i * bk).astype(jnp.float32)

    cw_max = 512
    for c0 in range(0, bk, cw_max):
        cw = min(cw_max, bk - c0)
        kc = k[c0:c0 + cw, :]
        mc = jax.lax.dot_general(
            q, kc, (((1,), (1,)), ((), ())),
            preferred_element_type=jnp.float32,
        )
        ksq_c = jnp.sum(kc * kc, axis=1)[None, :]
        bases = list(range(0, cw - 128, 128)) + [cw - 128]
        for b in bases:
            dj = (q_sq + ksq_c[:, b:b + 128]) - 2.0 * mc[:, b:b + 128]
            take = dj < val
            val = jnp.where(take, dj, val)
            base = jnp.where(take, off + jnp.float32(c0 + b), base)
    sval[...] = val
    sbase[...] = base

    @pl.when(i == nb - 1)
    def _final():
        rm = jnp.min(val, axis=1, keepdims=True)
        lane = jax.lax.broadcasted_iota(
            jnp.int32, (nq, 128), 1).astype(jnp.float32)
        cand = jnp.where(val == rm, base + lane, jnp.float32(2 * nb * bk))
        ri = jnp.min(cand, axis=1, keepdims=True)
        ix = jnp.where(rm > _THRESHOLD, jnp.float32(-1), ri)
        min_ref[...] = rm.reshape(nq)
        idx_ref[...] = ix.reshape(nq).astype(jnp.int32)


def kernel(source_embs, embeddings):
    q, d_dim = source_embs.shape
    n_k, _ = embeddings.shape
    bk = 2000
    assert n_k % bk == 0
    nb = n_k // bk

    body = functools.partial(_nn_body, bk=bk, nb=nb)
    idx1, min1 = pl.pallas_call(
        body,
        grid=(nb,),
        in_specs=[
            pl.BlockSpec((q, d_dim), lambda i: (0, 0)),
            pl.BlockSpec((bk, d_dim), lambda i: (i, 0)),
        ],
        out_specs=[
            pl.BlockSpec((q,), lambda i: (0,)),
            pl.BlockSpec((q,), lambda i: (0,)),
        ],
        out_shape=[
            jax.ShapeDtypeStruct((q,), jnp.int32),
            jax.ShapeDtypeStruct((q,), jnp.float32),
        ],
        scratch_shapes=[
            pltpu.VMEM((q, 128), jnp.float32),
            pltpu.VMEM((q, 128), jnp.float32),
            pltpu.VMEM((q, 1), jnp.float32),
        ],
        compiler_params=pltpu.CompilerParams(
            dimension_semantics=("arbitrary",),
        ),
    )(source_embs, embeddings)
    return (idx1, min1)


# speedup vs baseline: 1.0161x; 1.0161x over previous
import functools

import jax
import jax.numpy as jnp
from jax.experimental import pallas as pl
from jax.experimental.pallas import tpu as pltpu

_THRESHOLD = 1.5


def _nn_body(q_ref, k_ref, idx_ref, min_ref, sval, sbase, sqsq, *, bk, nb):
    i = pl.program_id(0)
    q = q_ref[...]
    k = k_ref[...]
    nq = q.shape[0]

    @pl.when(i == 0)
    def _qsq():
        sqsq[...] = jnp.sum(q * q, axis=1, keepdims=True)

    q_sq = sqsq[...]
    k_sq = jnp.sum(k * k, axis=1)[None, :]

    val = jnp.where(i == 0, jnp.inf, sval[...])
    base = jnp.where(i == 0, jnp.float32(0), sbase[...])
    off = (i * bk).astype(jnp.float32)

    cw_max = 256
    for c0 in range(0, bk, cw_max):
        cw = min(cw_max, bk - c0)
        mc = jax.lax.dot_general(
            q, k[c0:c0 + cw, :], (((1,), (1,)), ((), ())),
            preferred_element_type=jnp.float32,
        )
        ksq_c = k_sq[:, c0:c0 + cw]
        bases = list(range(0, cw - 128, 128)) + [cw - 128]
        for b in bases:
            dj = (q_sq + ksq_c[:, b:b + 128]) - 2.0 * mc[:, b:b + 128]
            take = dj < val
            val = jnp.where(take, dj, val)
            base = jnp.where(take, off + jnp.float32(c0 + b), base)
    sval[...] = val
    sbase[...] = base

    @pl.when(i == nb - 1)
    def _final():
        rm = jnp.min(val, axis=1, keepdims=True)
        lane = jax.lax.broadcasted_iota(
            jnp.int32, (nq, 128), 1).astype(jnp.float32)
        cand = jnp.where(val == rm, base + lane, jnp.float32(2 * nb * bk))
        ri = jnp.min(cand, axis=1, keepdims=True)
        ix = jnp.where(rm > _THRESHOLD, jnp.float32(-1), ri)
        min_ref[...] = rm.reshape(nq)
        idx_ref[...] = ix.reshape(nq).astype(jnp.int32)


def kernel(source_embs, embeddings):
    q, d_dim = source_embs.shape
    n_k, _ = embeddings.shape
    bk = 2000
    assert n_k % bk == 0
    nb = n_k // bk

    body = functools.partial(_nn_body, bk=bk, nb=nb)
    idx1, min1 = pl.pallas_call(
        body,
        grid=(nb,),
        in_specs=[
            pl.BlockSpec((q, d_dim), lambda i: (0, 0)),
            pl.BlockSpec((bk, d_dim), lambda i: (i, 0)),
        ],
        out_specs=[
            pl.BlockSpec((q,), lambda i: (0,)),
            pl.BlockSpec((q,), lambda i: (0,)),
        ],
        out_shape=[
            jax.ShapeDtypeStruct((q,), jnp.int32),
            jax.ShapeDtypeStruct((q,), jnp.float32),
        ],
        scratch_shapes=[
            pltpu.VMEM((q, 128), jnp.float32),
            pltpu.VMEM((q, 128), jnp.float32),
            pltpu.VMEM((q, 1), jnp.float32),
        ],
        compiler_params=pltpu.CompilerParams(
            dimension_semantics=("arbitrary",),
        ),
    )(source_embs, embeddings)
    return (idx1, min1)
